# baseline (speedup 1.0000x reference)
import functools

import jax
import jax.numpy as jnp
from jax import lax
from jax.experimental import pallas as pl
from jax.experimental.pallas import tpu as pltpu
from jax.experimental.pallas import tpu_sc as plsc

N = 10000
E = 320000
D_IN = 128
H = 64
OUT = 32

NC = 2
NS = 16
NW = NC * NS
CH = 128
NCHUNK = E // CH
T = NCHUNK // NW
XTRA = NCHUNK - T * NW
NB = 4
RND = T // NB
NPAD = 10240
STRIPE = NPAD // NS

_EPS = 1e-5


def _make_sc_agg(width, with_deg):
    mesh = plsc.VectorSubcoreMesh(core_axis_name="c", subcore_axis_name="s")

    out_type = [jax.ShapeDtypeStruct((NC, NPAD, width), jnp.float32)]
    scratch = [
        pltpu.VMEM((T * CH,), jnp.int32),
        pltpu.VMEM((T * CH,), jnp.int32),
        pltpu.VMEM((CH,), jnp.int32),
        pltpu.VMEM((CH,), jnp.int32),
    ] + [pltpu.VMEM((CH, width), jnp.float32) for _ in range(NB)] + [
        pltpu.VMEM_SHARED((NPAD, width), jnp.float32),
    ] + [pltpu.SemaphoreType.DMA for _ in range(2 * NB)]
    if with_deg:
        out_type.append(jax.ShapeDtypeStruct((NC, NPAD, 16), jnp.float32))
        scratch += [
            pltpu.VMEM((CH, 16), jnp.float32),
            pltpu.VMEM_SHARED((NPAD, 16), jnp.float32),
        ] + [pltpu.SemaphoreType.DMA for _ in range(NB)]

    @functools.partial(pl.kernel, mesh=mesh, out_type=out_type,
                       scratch_types=scratch,
                       compiler_params=pltpu.CompilerParams(
                           use_tc_tiling_on_sc=False))
    def body(*refs):
        if with_deg:
            (p_hbm, e_hbm, z_hbm, zd_hbm, s_out, d_out,
             src_v, dst_v, xsrc_v, xdst_v, *rest) = refs
            bufs = rest[:NB]
            acc = rest[NB]
            gsems = rest[NB + 1:2 * NB + 1]
            ssems = rest[2 * NB + 1:3 * NB + 1]
            ones_v, dacc, *dsems = rest[3 * NB + 1:]
        else:
            (p_hbm, e_hbm, z_hbm, s_out,
             src_v, dst_v, xsrc_v, xdst_v, *rest) = refs
            bufs = rest[:NB]
            acc = rest[NB]
            gsems = rest[NB + 1:2 * NB + 1]
            ssems = rest[2 * NB + 1:3 * NB + 1]

        c = lax.axis_index("c")
        s = lax.axis_index("s")
        wid = c * NS + s
        r0 = s * STRIPE

        e0 = wid * (T * CH)
        pltpu.sync_copy(e_hbm.at[0, pl.ds(e0, T * CH)], src_v)
        pltpu.sync_copy(e_hbm.at[1, pl.ds(e0, T * CH)], dst_v)

        @pl.when(wid < XTRA)
        def _stage_extra():
            x0 = NW * T * CH + wid * CH
            pltpu.sync_copy(e_hbm.at[0, pl.ds(x0, CH)], xsrc_v)
            pltpu.sync_copy(e_hbm.at[1, pl.ds(x0, CH)], xdst_v)

        pltpu.sync_copy(z_hbm.at[pl.ds(r0, STRIPE)], acc.at[pl.ds(r0, STRIPE)])
        if with_deg:
            pltpu.sync_copy(zd_hbm.at[pl.ds(r0, STRIPE)],
                            dacc.at[pl.ds(r0, STRIPE)])

            def fill(i, _):
                ones_v[i, :] = jnp.ones((16,), jnp.float32)
                return 0
            lax.fori_loop(0, CH, fill, 0)
        plsc.subcore_barrier()

        def gstart(j, b):
            pltpu.async_copy(p_hbm.at[src_v.at[pl.ds(j * CH, CH)]],
                             bufs[b], gsems[b])

        def gwait(b):
            pltpu.make_async_copy(p_hbm.at[src_v.at[pl.ds(0, CH)]], bufs[b],
                                  gsems[b]).wait()

        def sstart(j, b):
            pltpu.async_copy(bufs[b], acc.at[dst_v.at[pl.ds(j * CH, CH)]],
                             ssems[b], add=True)
            if with_deg:
                pltpu.async_copy(ones_v, dacc.at[dst_v.at[pl.ds(j * CH, CH)]],
                                 dsems[b], add=True)

        def swait(b):
            pltpu.make_async_copy(bufs[b], acc.at[dst_v.at[pl.ds(0, CH)]],
                                  ssems[b]).wait()
            if with_deg:
                pltpu.make_async_copy(ones_v,
                                      dacc.at[dst_v.at[pl.ds(0, CH)]],
                                      dsems[b]).wait()

        for b in range(NB):
            gstart(b, b)

        def rnd(r, _):
            base = r * NB
            for b in range(NB):
                gwait(b)
                sstart(base + b, b)
            for b in range(NB):
                swait(b)
                gstart(base + NB + b, b)
            return 0
        lax.fori_loop(0, RND - 1, rnd, 0)

        for b in range(NB):
            gwait(b)
            sstart(NB * (RND - 1) + b, b)
        TAIL = T - NB * RND
        for t in range(TAIL):
            swait(t)
            gstart(NB * RND + t, t)
        for b in range(TAIL, NB):
            swait(b)
        for t in range(TAIL):
            gwait(t)
            sstart(NB * RND + t, t)
            swait(t)

        @pl.when(wid < XTRA)
        def _extra_chunk():
            pltpu.async_copy(p_hbm.at[xsrc_v], bufs[0], gsems[0])
            pltpu.make_async_copy(p_hbm.at[xsrc_v], bufs[0],
                                  gsems[0]).wait()
            pltpu.sync_copy(bufs[0], acc.at[xdst_v], add=True)
            if with_deg:
                pltpu.sync_copy(ones_v, dacc.at[xdst_v], add=True)

        plsc.subcore_barrier()

        pltpu.sync_copy(acc.at[pl.ds(r0, STRIPE)],
                        s_out.at[c, pl.ds(r0, STRIPE)])
        if with_deg:
            pltpu.sync_copy(dacc.at[pl.ds(r0, STRIPE)],
                            d_out.at[c, pl.ds(r0, STRIPE)])

    return body


_sc_agg_deg = _make_sc_agg(H, True)
_sc_agg_h = _make_sc_agg(H, False)
_sc_agg_out = _make_sc_agg(OUT, False)


def _tc_pre_body(x_ref, wl_ref, wr_ref, p_ref, r_ref):
    x = x_ref[...]
    p_ref[...] = jnp.dot(x, wl_ref[...], preferred_element_type=jnp.float32)
    r_ref[...] = jnp.dot(x, wr_ref[...], preferred_element_type=jnp.float32)


GB = 1000
G = N // GB


def _tc_pre(x, wl, wr):
    ho = wl.shape[1]
    return pl.pallas_call(
        _tc_pre_body,
        grid=(G,),
        in_specs=[pl.BlockSpec((GB, D_IN), lambda g: (g, 0)),
                  pl.BlockSpec((D_IN, ho), lambda g: (0, 0)),
                  pl.BlockSpec((D_IN, ho), lambda g: (0, 0))],
        out_specs=[pl.BlockSpec((GB, ho), lambda g: (g, 0)),
                   pl.BlockSpec((GB, ho), lambda g: (0, 0) if False else (g, 0))],
        out_shape=[jax.ShapeDtypeStruct((N, ho), jnp.float32),
                   jax.ShapeDtypeStruct((N, ho), jnp.float32)],
    )(x, wl, wr)


def _z_block(s_ref, deg_ref, r_ref, b_ref):
    ssum = s_ref[0] + s_ref[1]
    deg = (deg_ref[0] + deg_ref[1])[:, 0:1]
    agg = ssum / jnp.maximum(deg, 1.0)
    return agg + r_ref[...] + b_ref[...]


def _stats_pass(p, z, stat_scr):
    @pl.when(p == 0)
    def _():
        g = pl.program_id(1)

        @pl.when(g == 0)
        def _():
            stat_scr[...] = jnp.zeros_like(stat_scr)
        w = z.shape[1]
        stat_scr[0:1, :w] += jnp.sum(z, axis=0, keepdims=True)
        stat_scr[1:2, :w] += jnp.sum(z * z, axis=0, keepdims=True)


def _norm_block(z, stat_scr, g_ref, bb_ref):
    w = z.shape[1]
    mu = stat_scr[0:1, :w] / N
    var = stat_scr[1:2, :w] / N - mu * mu
    return (z - mu) * lax.rsqrt(var + _EPS) * g_ref[...] + bb_ref[...]


def _tc_mid_body(s_ref, deg_ref, r_ref, b_ref, g_ref, bb_ref,
                 wl_ref, wr_ref, p_ref, rn_ref, stat_scr):
    p = pl.program_id(0)
    z = _z_block(s_ref, deg_ref, r_ref, b_ref)
    _stats_pass(p, z, stat_scr)

    @pl.when(p == 1)
    def _():
        h = jnp.maximum(_norm_block(z, stat_scr, g_ref, bb_ref), 0.0)
        p_ref[...] = jnp.dot(h, wl_ref[...],
                             preferred_element_type=jnp.float32)
        rn_ref[...] = jnp.dot(h, wr_ref[...],
                              preferred_element_type=jnp.float32)


def _mid_specs(w, wo1, wo2):
    return [pl.BlockSpec((NC, GB, w), lambda p, g: (0, g, 0)),
            pl.BlockSpec((NC, GB, 16), lambda p, g: (0, g, 0)),
            pl.BlockSpec((GB, w), lambda p, g: (g, 0)),
            pl.BlockSpec((1, w), lambda p, g: (0, 0)),
            pl.BlockSpec((1, w), lambda p, g: (0, 0)),
            pl.BlockSpec((1, w), lambda p, g: (0, 0)),
            pl.BlockSpec((w, wo1), lambda p, g: (0, 0)),
            pl.BlockSpec((w, wo2), lambda p, g: (0, 0))]


def _tc_mid(s_parts, deg_parts, r, b, g, bb, wl, wr):
    w = r.shape[1]
    return pl.pallas_call(
        _tc_mid_body,
        grid=(2, G),
        in_specs=_mid_specs(w, wl.shape[1], wr.shape[1]),
        out_specs=[pl.BlockSpec((GB, wl.shape[1]), lambda p, g: (g, 0)),
                   pl.BlockSpec((GB, wr.shape[1]), lambda p, g: (g, 0))],
        out_shape=[jax.ShapeDtypeStruct((N, wl.shape[1]), jnp.float32),
                   jax.ShapeDtypeStruct((N, wr.shape[1]), jnp.float32)],
        scratch_shapes=[pltpu.VMEM((2, 128), jnp.float32)],
    )(s_parts, deg_parts, r, b, g, bb, wl, wr)


def _tc_final_body(s_ref, deg_ref, r_ref, b_ref, g_ref, bb_ref,
                   w1_ref, b1_ref, w2_ref, b2_ref, pred_ref, emb_ref,
                   stat_scr):
    p = pl.program_id(0)
    z = _z_block(s_ref, deg_ref, r_ref, b_ref)
    _stats_pass(p, z, stat_scr)

    @pl.when(p == 1)
    def _():
        emb = _norm_block(z, stat_scr, g_ref, bb_ref)
        emb_ref[...] = emb
        hid = jnp.maximum(
            jnp.dot(emb, w1_ref[...], preferred_element_type=jnp.float32)
            + b1_ref[...], 0.0)
        logit = jnp.dot(hid, w2_ref[...],
                        preferred_element_type=jnp.float32) + b2_ref[...]
        pred_ref[...] = jax.nn.sigmoid(logit)


def _tc_final(s_parts, deg_parts, r, b, g, bb, w1, b1, w2, b2):
    return pl.pallas_call(
        _tc_final_body,
        grid=(2, G),
        in_specs=_mid_specs(OUT, H, 1)[:7] + [
            pl.BlockSpec((1, H), lambda p, g: (0, 0)),
            pl.BlockSpec((H, 1), lambda p, g: (0, 0)),
            pl.BlockSpec((1, 1), lambda p, g: (0, 0))],
        out_specs=[pl.BlockSpec((GB, 1), lambda p, g: (g, 0)),
                   pl.BlockSpec((GB, OUT), lambda p, g: (g, 0))],
        out_shape=[jax.ShapeDtypeStruct((N, 1), jnp.float32),
                   jax.ShapeDtypeStruct((N, OUT), jnp.float32)],
        scratch_shapes=[pltpu.VMEM((2, 128), jnp.float32)],
    )(s_parts, deg_parts, r, b, g, bb, w1, b1, w2, b2)


def kernel(x, edge_index, W_l0, W_r0, b0, bn_g0, bn_b0, W_l1, W_r1, b1,
           bn_g1, bn_b1, W_l2, W_r2, b2, bn_g2, bn_b2, cls_W1, cls_b1,
           cls_W2, cls_b2):
    z64 = jnp.zeros((NPAD, H), jnp.float32)
    z32 = jnp.zeros((NPAD, OUT), jnp.float32)
    z16 = jnp.zeros((NPAD, 16), jnp.float32)

    p0, r0 = _tc_pre(x, W_l0, W_r0)
    s0, degp = _sc_agg_deg(p0, edge_index, z64, z16)
    p1, r1 = _tc_mid(s0, degp, r0, b0.reshape(1, H), bn_g0.reshape(1, H),
                     bn_b0.reshape(1, H), W_l1, W_r1)
    (s1,) = _sc_agg_h(p1, edge_index, z64)
    p2, r2 = _tc_mid(s1, degp, r1, b1.reshape(1, H), bn_g1.reshape(1, H),
                     bn_b1.reshape(1, H), W_l2, W_r2)
    (s2,) = _sc_agg_out(p2, edge_index, z32)
    pred, emb = _tc_final(s2, degp, r2, b2.reshape(1, OUT),
                          bn_g2.reshape(1, OUT), bn_b2.reshape(1, OUT),
                          cls_W1, cls_b1.reshape(1, H), cls_W2,
                          cls_b2.reshape(1, 1))
    return (pred, emb)


# speedup vs baseline: 1.1310x; 1.1310x over previous
import functools

import jax
import jax.numpy as jnp
from jax import lax
from jax.experimental import pallas as pl
from jax.experimental.pallas import tpu as pltpu
from jax.experimental.pallas import tpu_sc as plsc

N = 10000
E = 320000
D_IN = 128
H = 64
OUT = 32

NC = 2
NS = 16
NW = NC * NS
CH = 128
NCHUNK = E // CH
T = NCHUNK // NW
XTRA = NCHUNK - T * NW
NB = 4
RND = T // NB
NPAD = 10240
STRIPE = NPAD // NS

_EPS = 1e-5


def _make_sc_agg(width, with_deg):
    mesh = plsc.VectorSubcoreMesh(core_axis_name="c", subcore_axis_name="s")

    out_type = [jax.ShapeDtypeStruct((NC, NPAD, width), jnp.float32)]
    scratch = [
        pltpu.VMEM((T * CH,), jnp.int32),
        pltpu.VMEM((T * CH,), jnp.int32),
        pltpu.VMEM((CH,), jnp.int32),
        pltpu.VMEM((CH,), jnp.int32),
    ] + [pltpu.VMEM((CH, width), jnp.float32) for _ in range(NB)] + [
        pltpu.VMEM_SHARED((NPAD, width), jnp.float32),
    ] + [pltpu.SemaphoreType.DMA for _ in range(2 * NB)]
    if with_deg:
        out_type.append(jax.ShapeDtypeStruct((NC, NPAD, 16), jnp.float32))
        scratch += [
            pltpu.VMEM((CH, 16), jnp.float32),
            pltpu.VMEM_SHARED((NPAD, 16), jnp.float32),
        ] + [pltpu.SemaphoreType.DMA for _ in range(NB)]

    @functools.partial(pl.kernel, mesh=mesh, out_type=out_type,
                       scratch_types=scratch,
                       compiler_params=pltpu.CompilerParams(
                           use_tc_tiling_on_sc=False))
    def body(*refs):
        if with_deg:
            (p_hbm, e_hbm, z_hbm, zd_hbm, s_out, d_out,
             src_v, dst_v, xsrc_v, xdst_v, *rest) = refs
            bufs = rest[:NB]
            acc = rest[NB]
            gsems = rest[NB + 1:2 * NB + 1]
            ssems = rest[2 * NB + 1:3 * NB + 1]
            ones_v, dacc, *dsems = rest[3 * NB + 1:]
        else:
            (p_hbm, e_hbm, z_hbm, s_out,
             src_v, dst_v, xsrc_v, xdst_v, *rest) = refs
            bufs = rest[:NB]
            acc = rest[NB]
            gsems = rest[NB + 1:2 * NB + 1]
            ssems = rest[2 * NB + 1:3 * NB + 1]

        c = lax.axis_index("c")
        s = lax.axis_index("s")
        wid = c * NS + s
        r0 = s * STRIPE

        e0 = wid * (T * CH)
        pltpu.sync_copy(e_hbm.at[0, pl.ds(e0, T * CH)], src_v)
        pltpu.sync_copy(e_hbm.at[1, pl.ds(e0, T * CH)], dst_v)

        @pl.when(wid < XTRA)
        def _stage_extra():
            x0 = NW * T * CH + wid * CH
            pltpu.sync_copy(e_hbm.at[0, pl.ds(x0, CH)], xsrc_v)
            pltpu.sync_copy(e_hbm.at[1, pl.ds(x0, CH)], xdst_v)

        pltpu.sync_copy(z_hbm.at[pl.ds(r0, STRIPE)], acc.at[pl.ds(r0, STRIPE)])
        if with_deg:
            pltpu.sync_copy(zd_hbm.at[pl.ds(r0, STRIPE)],
                            dacc.at[pl.ds(r0, STRIPE)])

            def fill(i, _):
                ones_v[i, :] = jnp.ones((16,), jnp.float32)
                return 0
            lax.fori_loop(0, CH, fill, 0)
        plsc.subcore_barrier()

        def gstart(j, b):
            pltpu.async_copy(p_hbm.at[src_v.at[pl.ds(j * CH, CH)]],
                             bufs[b], gsems[b])

        def gwait(b):
            pltpu.make_async_copy(p_hbm.at[src_v.at[pl.ds(0, CH)]], bufs[b],
                                  gsems[b]).wait()

        def sstart(j, b):
            pltpu.async_copy(bufs[b], acc.at[dst_v.at[pl.ds(j * CH, CH)]],
                             ssems[b], add=True)
            if with_deg:
                pltpu.async_copy(ones_v, dacc.at[dst_v.at[pl.ds(j * CH, CH)]],
                                 dsems[b], add=True)

        def swait(b):
            pltpu.make_async_copy(bufs[b], acc.at[dst_v.at[pl.ds(0, CH)]],
                                  ssems[b]).wait()
            if with_deg:
                pltpu.make_async_copy(ones_v,
                                      dacc.at[dst_v.at[pl.ds(0, CH)]],
                                      dsems[b]).wait()

        for b in range(NB):
            gstart(b, b)

        def rnd(r, _):
            base = r * NB
            for b in range(NB):
                gwait(b)
                sstart(base + b, b)
            for b in range(NB):
                swait(b)
                gstart(base + NB + b, b)
            return 0
        lax.fori_loop(0, RND - 1, rnd, 0)

        for b in range(NB):
            gwait(b)
            sstart(NB * (RND - 1) + b, b)
        TAIL = T - NB * RND
        for t in range(TAIL):
            swait(t)
            gstart(NB * RND + t, t)
        for b in range(TAIL, NB):
            swait(b)
        for t in range(TAIL):
            gwait(t)
            sstart(NB * RND + t, t)
            swait(t)

        @pl.when(wid < XTRA)
        def _extra_chunk():
            pltpu.async_copy(p_hbm.at[xsrc_v], bufs[0], gsems[0])
            pltpu.make_async_copy(p_hbm.at[xsrc_v], bufs[0],
                                  gsems[0]).wait()
            pltpu.sync_copy(bufs[0], acc.at[xdst_v], add=True)
            if with_deg:
                pltpu.sync_copy(ones_v, dacc.at[xdst_v], add=True)

        plsc.subcore_barrier()

        pltpu.sync_copy(acc.at[pl.ds(r0, STRIPE)],
                        s_out.at[c, pl.ds(r0, STRIPE)])
        if with_deg:
            pltpu.sync_copy(dacc.at[pl.ds(r0, STRIPE)],
                            d_out.at[c, pl.ds(r0, STRIPE)])

    return body


_sc_agg_deg = _make_sc_agg(H, True)
_sc_agg_h = _make_sc_agg(H, False)
_sc_agg_out = _make_sc_agg(OUT, False)


def _tc_pre_body(x_ref, wl_ref, wr_ref, p_ref, r_ref):
    x = x_ref[...]
    p_ref[...] = jnp.dot(x, wl_ref[...], preferred_element_type=jnp.float32)
    r_ref[...] = jnp.dot(x, wr_ref[...], preferred_element_type=jnp.float32)


def _tc_pre(x, wl, wr):
    return pl.pallas_call(
        _tc_pre_body,
        out_shape=[jax.ShapeDtypeStruct((N, wl.shape[1]), jnp.float32),
                   jax.ShapeDtypeStruct((N, wr.shape[1]), jnp.float32)],
    )(x, wl, wr)


def _bn_from_parts(s_ref, deg_ref, r_ref, b_ref, g_ref, bb_ref, relu):
    ssum = (s_ref[0] + s_ref[1])[:N]
    deg = (deg_ref[0] + deg_ref[1])[:N, 0:1]
    agg = ssum / jnp.maximum(deg, 1.0)
    z = agg + r_ref[...] + b_ref[...]
    mu = jnp.mean(z, axis=0, keepdims=True)
    var = jnp.mean((z - mu) * (z - mu), axis=0, keepdims=True)
    h = (z - mu) * lax.rsqrt(var + _EPS) * g_ref[...] + bb_ref[...]
    if relu:
        h = jnp.maximum(h, 0.0)
    return h


def _tc_mid_body(s_ref, deg_ref, r_ref, b_ref, g_ref, bb_ref,
                 wl_ref, wr_ref, p_ref, rn_ref):
    h = _bn_from_parts(s_ref, deg_ref, r_ref, b_ref, g_ref, bb_ref, True)
    p_ref[...] = jnp.dot(h, wl_ref[...], preferred_element_type=jnp.float32)
    rn_ref[...] = jnp.dot(h, wr_ref[...], preferred_element_type=jnp.float32)


def _tc_mid(s_parts, deg_parts, r, b, g, bb, wl, wr):
    return pl.pallas_call(
        _tc_mid_body,
        out_shape=[jax.ShapeDtypeStruct((N, wl.shape[1]), jnp.float32),
                   jax.ShapeDtypeStruct((N, wr.shape[1]), jnp.float32)],
    )(s_parts, deg_parts, r, b, g, bb, wl, wr)


def _tc_final_body(s_ref, deg_ref, r_ref, b_ref, g_ref, bb_ref,
                   w1_ref, b1_ref, w2_ref, b2_ref, pred_ref, emb_ref):
    emb = _bn_from_parts(s_ref, deg_ref, r_ref, b_ref, g_ref, bb_ref, False)
    emb_ref[...] = emb
    hid = jnp.maximum(
        jnp.dot(emb, w1_ref[...], preferred_element_type=jnp.float32)
        + b1_ref[...], 0.0)
    logit = jnp.dot(hid, w2_ref[...], preferred_element_type=jnp.float32) \
        + b2_ref[...]
    pred_ref[...] = jax.nn.sigmoid(logit)


def _tc_final(s_parts, deg_parts, r, b, g, bb, w1, b1, w2, b2):
    return pl.pallas_call(
        _tc_final_body,
        out_shape=[jax.ShapeDtypeStruct((N, 1), jnp.float32),
                   jax.ShapeDtypeStruct((N, OUT), jnp.float32)],
    )(s_parts, deg_parts, r, b, g, bb, w1, b1, w2, b2)


def kernel(x, edge_index, W_l0, W_r0, b0, bn_g0, bn_b0, W_l1, W_r1, b1,
           bn_g1, bn_b1, W_l2, W_r2, b2, bn_g2, bn_b2, cls_W1, cls_b1,
           cls_W2, cls_b2):
    z64 = jnp.zeros((NPAD, H), jnp.float32)
    z32 = jnp.zeros((NPAD, OUT), jnp.float32)
    z16 = jnp.zeros((NPAD, 16), jnp.float32)

    p0, r0 = _tc_pre(x, W_l0, W_r0)
    s0, degp = _sc_agg_deg(p0, edge_index, z64, z16)
    p1, r1 = _tc_mid(s0, degp, r0, b0.reshape(1, H), bn_g0.reshape(1, H),
                     bn_b0.reshape(1, H), W_l1, W_r1)
    (s1,) = _sc_agg_h(p1, edge_index, z64)
    p2, r2 = _tc_mid(s1, degp, r1, b1.reshape(1, H), bn_g1.reshape(1, H),
                     bn_b1.reshape(1, H), W_l2, W_r2)
    (s2,) = _sc_agg_out(p2, edge_index, z32)
    pred, emb = _tc_final(s2, degp, r2, b2.reshape(1, OUT),
                          bn_g2.reshape(1, OUT), bn_b2.reshape(1, OUT),
                          cls_W1, cls_b1.reshape(1, H), cls_W2,
                          cls_b2.reshape(1, 1))
    return (pred, emb)
